# baseline (speedup 1.0000x reference)
import functools

import jax
import jax.numpy as jnp
from jax import lax
from jax.experimental import layout as jex_layout
from jax.experimental import pallas as pl
from jax.experimental.pallas import tpu as pltpu
from jax.experimental.pallas import tpu_sc as plsc

N = 1048576
NC = 2
NS = 16
NW = NC * NS
C = N // NW
B = 512
SC_ROWS = 2048
NSUB = C // SC_ROWS
GPS = SC_ROWS // 16
RING = 4096
ZB = 256
NZB = C // ZB

_mesh = plsc.VectorSubcoreMesh(
    core_axis_name="c", subcore_axis_name="s", num_cores=NC, num_subcores=NS
)
_params = pltpu.CompilerParams(
    needs_layout_passes=False, use_tc_tiling_on_sc=False
)


def _wid():
    return lax.axis_index("s") * NC + lax.axis_index("c")


@functools.partial(
    pl.kernel,
    out_type=jax.ShapeDtypeStruct((NW * 16,), jnp.int32),
    mesh=_mesh,
    compiler_params=_params,
    scratch_types=[
        [pltpu.VMEM((SC_ROWS * 2,), jnp.float32)] * 2,
        pltpu.VMEM((16,), jnp.int32),
        [pltpu.SemaphoreType.DMA] * 2,
    ],
)
def _count_kernel(feat_hbm, counts_hbm, featp_v, cnt_v, sems):
    w = _wid()
    base = w * C
    lanes = lax.iota(jnp.int32, 16)

    def start_load(s):
        b = s % 2
        return pltpu.async_copy(
            feat_hbm.at[pl.ds(2 * (base + s * SC_ROWS), 2 * SC_ROWS)],
            featp_v[b],
            sems[b],
        )

    pending = start_load(0)
    cnt = jnp.zeros((16,), jnp.int32)
    for s in range(NSUB):
        pending.wait()
        if s + 1 < NSUB:
            pending = start_load(s + 1)
        featp = featp_v[s % 2]

        def body(i, c):
            off = 256 * (i // 8) + 128 + 16 * (i % 8)
            col1 = featp[pl.ds(off, 16)]
            return c + plsc.all_reduce_population_count(col1 > 0.5)

        cnt = lax.fori_loop(0, GPS, body, cnt, unroll=8)
    cnt_v[...] = cnt
    pltpu.sync_copy(cnt_v, counts_hbm.at[pl.ds(w * 16, 16)])


@functools.partial(
    pl.kernel,
    out_type=(
        jax.ShapeDtypeStruct((N, 2), jnp.float32),
        jax.ShapeDtypeStruct((N, 4), jnp.int32),
        jax.ShapeDtypeStruct((16,), jnp.int32),
    ),
    mesh=_mesh,
    compiler_params=_params,
    scratch_types=[
        pltpu.VMEM((NW * 16,), jnp.int32),
        [pltpu.VMEM((SC_ROWS * 2,), jnp.float32)] * 2,
        [pltpu.VMEM((SC_ROWS * 4,), jnp.int32)] * 2,
        pltpu.VMEM((RING, 2), jnp.float32),
        pltpu.VMEM((RING, 4), jnp.int32),
        pltpu.VMEM((ZB, 2), jnp.float32),
        pltpu.VMEM((ZB, 4), jnp.int32),
        pltpu.VMEM((16,), jnp.int32),
        [pltpu.SemaphoreType.DMA] * 2,
    ],
)
def _compact_kernel(
    feat_hbm, loc_hbm, counts_hbm, zerof_hbm, zeroi_hbm,
    outf_hbm, outl_hbm, nsel_hbm,
    counts_v, featp_v, locp_v, ringf_v, ringl_v, zf_v, zi_v, nsel_v, sems,
):
    w = _wid()
    base = w * C
    lanes = lax.iota(jnp.int32, 16)
    zeros_v = jnp.zeros((16,), jnp.int32)
    ones_v = jnp.ones((16,), jnp.int32)

    pltpu.sync_copy(counts_hbm, counts_v)
    pltpu.sync_copy(zerof_hbm, zf_v)
    pltpu.sync_copy(zeroi_hbm, zi_v)
    c0 = plsc.load_gather(counts_v, [lanes * 16])
    c1 = plsc.load_gather(counts_v, [(lanes + 16) * 16])
    cnt_w = jnp.sum(jnp.where(lanes == w, c0, 0)) + jnp.sum(
        jnp.where(lanes + 16 == w, c1, 0)
    )
    w_off = jnp.sum(jnp.where(lanes < w, c0, 0)) + jnp.sum(
        jnp.where(lanes + 16 < w, c1, 0)
    )
    nsel = jnp.sum(c0) + jnp.sum(c1)

    def start_load(s):
        b = s % 2
        row = base + s * SC_ROWS
        return (
            pltpu.async_copy(
                feat_hbm.at[pl.ds(2 * row, 2 * SC_ROWS)], featp_v[b], sems[b]
            ),
            pltpu.async_copy(
                loc_hbm.at[pl.ds(4 * row, 4 * SC_ROWS)], locp_v[b], sems[b]
            ),
        )

    pending = start_load(0)
    cntvec = jnp.zeros((16,), jnp.int32)
    flushed = jnp.int32(0)

    for s in range(NSUB):
        for cp in pending:
            cp.wait()
        if s + 1 < NSUB:
            pending = start_load(s + 1)
        b = s % 2
        featp = featp_v[b]
        locp = locp_v[b]

        def group(g, cnt):
            foff = 256 * (g // 8) + 16 * (g % 8)
            loff = 512 * (g // 8) + 16 * (g % 8)
            f1 = featp[pl.ds(foff + 128, 16)]
            m = f1 > 0.5
            pos = cnt + plsc.cumsum(m.astype(jnp.int32)) - 1
            rp = pos & (RING - 1)
            f0 = featp[pl.ds(foff, 16)]
            plsc.store_scatter(ringf_v, [rp, zeros_v], f0, mask=m)
            plsc.store_scatter(ringf_v, [rp, ones_v], f1, mask=m)
            for c in range(4):
                cc = jnp.full((16,), c, jnp.int32)
                lv = locp[pl.ds(loff + 128 * c, 16)]
                plsc.store_scatter(ringl_v, [rp, cc], lv, mask=m)
            return cnt + plsc.all_reduce_population_count(m)

        cntvec = lax.fori_loop(0, GPS, group, cntvec, unroll=4)

        cnt_s = jnp.sum(jnp.where(lanes == 0, cntvec, 0))
        nblk = (cnt_s - flushed) // B

        def flush(k, fl):
            roff = fl & (RING - 1)
            pltpu.sync_copy(
                ringf_v.at[pl.ds(roff, B)], outf_hbm.at[pl.ds(w_off + fl, B)]
            )
            pltpu.sync_copy(
                ringl_v.at[pl.ds(roff, B)], outl_hbm.at[pl.ds(w_off + fl, B)]
            )
            return fl + B

        flushed = lax.fori_loop(0, nblk, flush, flushed)

    rem = cnt_w - flushed
    rbase = flushed & (RING - 1)
    dst = w_off + flushed
    off = jnp.int32(0)
    for sz in (256, 128, 64, 32, 16, 8, 4, 2, 1):
        part = rem & sz
        cur = off

        @pl.when(part > 0)
        def _copy(cur=cur, sz=sz):
            pltpu.sync_copy(
                ringf_v.at[pl.ds(rbase + cur, sz)],
                outf_hbm.at[pl.ds(dst + cur, sz)],
            )
            pltpu.sync_copy(
                ringl_v.at[pl.ds(rbase + cur, sz)],
                outl_hbm.at[pl.ds(dst + cur, sz)],
            )

        off = cur + part

    zb = jnp.maximum(0, jnp.minimum(NZB, (nsel - base + ZB - 1) // ZB))

    def zero_body(k, carry):
        row = base + k * ZB
        pltpu.sync_copy(zf_v, outf_hbm.at[pl.ds(row, ZB)])
        pltpu.sync_copy(zi_v, outl_hbm.at[pl.ds(row, ZB)])
        return carry

    lax.fori_loop(zb, NZB, zero_body, 0)

    gap = (ZB - (nsel & (ZB - 1))) & (ZB - 1)
    zoff = jnp.int32(0)
    for zsz in (128, 64, 32, 16, 8, 4, 2, 1):
        zpart = gap & zsz
        zcur = zoff

        @pl.when((zpart > 0) & (nsel >= base) & (nsel < base + C))
        def _zcopy(zcur=zcur, zsz=zsz):
            pltpu.sync_copy(
                zf_v.at[pl.ds(0, zsz)], outf_hbm.at[pl.ds(nsel + zcur, zsz)]
            )
            pltpu.sync_copy(
                zi_v.at[pl.ds(0, zsz)], outl_hbm.at[pl.ds(nsel + zcur, zsz)]
            )

        zoff = zcur + zpart

    @pl.when(w == 0)
    def _nsel():
        nsel_v[...] = jnp.full((16,), nsel, jnp.int32)
        pltpu.sync_copy(nsel_v, nsel_hbm)


TS = 2048
NTS = C // TS
TBL = TS // 128


@functools.partial(
    pl.kernel,
    out_type=(
        jax.ShapeDtypeStruct((2 * N,), jnp.float32),
        jax.ShapeDtypeStruct((4 * N,), jnp.int32),
    ),
    mesh=_mesh,
    compiler_params=_params,
    scratch_types=[
        pltpu.VMEM((TS * 2,), jnp.float32),
        pltpu.VMEM((TS * 2,), jnp.float32),
        pltpu.VMEM((TS * 4,), jnp.int32),
        pltpu.VMEM((TS * 4,), jnp.int32),
    ],
)
def _transpose_kernel(frm_hbm, lrm_hbm, fb_hbm, lb_hbm, fi_v, fo_v, li_v, lo_v):
    w = _wid()
    base = w * C
    lanes = lax.iota(jnp.int32, 16)

    for t in range(NTS):
        row = base + t * TS
        pltpu.sync_copy(frm_hbm.at[pl.ds(2 * row, 2 * TS)], fi_v)
        pltpu.sync_copy(lrm_hbm.at[pl.ds(4 * row, 4 * TS)], li_v)

        def tb(bl, carry):
            fb = 256 * bl
            lb = 512 * bl
            for j in range(8):
                r = 16 * j + lanes
                c0 = plsc.load_gather(fi_v, [fb + r * 2])
                c1 = plsc.load_gather(fi_v, [fb + r * 2 + 1])
                fo_v[pl.ds(fb + 16 * j, 16)] = c0
                fo_v[pl.ds(fb + 128 + 16 * j, 16)] = c1
                for c in range(4):
                    lv = plsc.load_gather(li_v, [lb + r * 4 + c])
                    lo_v[pl.ds(lb + 128 * c + 16 * j, 16)] = lv
            return carry

        lax.fori_loop(0, TBL, tb, 0)
        pltpu.sync_copy(fo_v, fb_hbm.at[pl.ds(2 * row, 2 * TS)])
        pltpu.sync_copy(lo_v, lb_hbm.at[pl.ds(4 * row, 4 * TS)])


@functools.lru_cache(maxsize=1)
def _jitted_kernel():
    dev = jax.devices()[0]
    sharding = jax.sharding.SingleDeviceSharding(dev)
    rm2 = jex_layout.Format(
        jex_layout.Layout(major_to_minor=(0, 1)), sharding
    )
    sc = jex_layout.Format(jex_layout.Layout(major_to_minor=()), sharding)
    return jax.jit(
        _kernel_impl,
        in_shardings=(rm2, rm2),
        out_shardings=(rm2, rm2, sc),
    )


def kernel(features, locations):
    return _jitted_kernel()(features, locations)


def _kernel_impl(features, locations):
    locations = locations.astype(jnp.int32)
    feat_b = features.reshape(N // 128, 128, 2).transpose(0, 2, 1).reshape(-1)
    loc_b = locations.reshape(N // 128, 128, 4).transpose(0, 2, 1).reshape(-1)
    counts = _count_kernel(feat_b)
    zerof = jnp.zeros((ZB, 2), jnp.float32)
    zeroi = jnp.zeros((ZB, 4), jnp.int32)
    outf, outl, nsel = _compact_kernel(
        feat_b, loc_b, counts, zerof, zeroi
    )
    fb, lb = _transpose_kernel(outf.reshape(-1), outl.reshape(-1))
    sel_f = fb.reshape(N // 128, 2, 128).transpose(0, 2, 1).reshape(N, 2)
    sel_l = lb.reshape(N // 128, 4, 128).transpose(0, 2, 1).reshape(N, 4)
    return sel_f, sel_l, nsel[0]


# speedup vs baseline: 1.0017x; 1.0017x over previous
import functools

import jax
import jax.numpy as jnp
from jax import lax
from jax.experimental import layout as jex_layout
from jax.experimental import pallas as pl
from jax.experimental.pallas import tpu as pltpu
from jax.experimental.pallas import tpu_sc as plsc

N = 1048576
NC = 2
NS = 16
NW = NC * NS
C = N // NW
B = 512
SC_ROWS = 2048
NSUB = C // SC_ROWS
GPS = SC_ROWS // 16
RING = 4096
ZB = 256
NZB = C // ZB

_mesh = plsc.VectorSubcoreMesh(
    core_axis_name="c", subcore_axis_name="s", num_cores=NC, num_subcores=NS
)
_params = pltpu.CompilerParams(
    needs_layout_passes=False, use_tc_tiling_on_sc=False
)


def _wid():
    return lax.axis_index("s") * NC + lax.axis_index("c")


@functools.partial(
    pl.kernel,
    out_type=jax.ShapeDtypeStruct((NW * 16,), jnp.int32),
    mesh=_mesh,
    compiler_params=_params,
    scratch_types=[
        [pltpu.VMEM((SC_ROWS * 2,), jnp.float32)] * 2,
        pltpu.VMEM((16,), jnp.int32),
        [pltpu.SemaphoreType.DMA] * 2,
    ],
)
def _count_kernel(feat_hbm, counts_hbm, featp_v, cnt_v, sems):
    w = _wid()
    base = w * C
    lanes = lax.iota(jnp.int32, 16)

    def start_load(s):
        b = s % 2
        return pltpu.async_copy(
            feat_hbm.at[pl.ds(2 * (base + s * SC_ROWS), 2 * SC_ROWS)],
            featp_v[b],
            sems[b],
        )

    pending = start_load(0)
    cnt = jnp.zeros((16,), jnp.int32)
    for s in range(NSUB):
        pending.wait()
        if s + 1 < NSUB:
            pending = start_load(s + 1)
        featp = featp_v[s % 2]

        def body(i, c):
            off = 256 * (i // 8) + 128 + 16 * (i % 8)
            col1 = featp[pl.ds(off, 16)]
            return c + plsc.all_reduce_population_count(col1 > 0.5)

        cnt = lax.fori_loop(0, GPS, body, cnt, unroll=8)
    cnt_v[...] = cnt
    pltpu.sync_copy(cnt_v, counts_hbm.at[pl.ds(w * 16, 16)])


@functools.partial(
    pl.kernel,
    out_type=(
        jax.ShapeDtypeStruct((N, 2), jnp.float32),
        jax.ShapeDtypeStruct((N, 4), jnp.int32),
        jax.ShapeDtypeStruct((16,), jnp.int32),
    ),
    mesh=_mesh,
    compiler_params=_params,
    scratch_types=[
        pltpu.VMEM((NW * 16,), jnp.int32),
        [pltpu.VMEM((SC_ROWS * 2,), jnp.float32)] * 2,
        [pltpu.VMEM((SC_ROWS * 4,), jnp.int32)] * 2,
        pltpu.VMEM((RING, 2), jnp.float32),
        pltpu.VMEM((RING, 4), jnp.int32),
        pltpu.VMEM((ZB, 2), jnp.float32),
        pltpu.VMEM((ZB, 4), jnp.int32),
        pltpu.VMEM((16,), jnp.int32),
        [pltpu.SemaphoreType.DMA] * 2,
    ],
)
def _compact_kernel(
    feat_hbm, loc_hbm, counts_hbm, zerof_hbm, zeroi_hbm,
    outf_hbm, outl_hbm, nsel_hbm,
    counts_v, featp_v, locp_v, ringf_v, ringl_v, zf_v, zi_v, nsel_v, sems,
):
    w = _wid()
    base = w * C
    lanes = lax.iota(jnp.int32, 16)
    zeros_v = jnp.zeros((16,), jnp.int32)
    ones_v = jnp.ones((16,), jnp.int32)

    pltpu.sync_copy(counts_hbm, counts_v)
    pltpu.sync_copy(zerof_hbm, zf_v)
    pltpu.sync_copy(zeroi_hbm, zi_v)
    c0 = plsc.load_gather(counts_v, [lanes * 16])
    c1 = plsc.load_gather(counts_v, [(lanes + 16) * 16])
    cnt_w = jnp.sum(jnp.where(lanes == w, c0, 0)) + jnp.sum(
        jnp.where(lanes + 16 == w, c1, 0)
    )
    w_off = jnp.sum(jnp.where(lanes < w, c0, 0)) + jnp.sum(
        jnp.where(lanes + 16 < w, c1, 0)
    )
    nsel = jnp.sum(c0) + jnp.sum(c1)

    def start_load(s):
        b = s % 2
        row = base + s * SC_ROWS
        return (
            pltpu.async_copy(
                feat_hbm.at[pl.ds(2 * row, 2 * SC_ROWS)], featp_v[b], sems[b]
            ),
            pltpu.async_copy(
                loc_hbm.at[pl.ds(4 * row, 4 * SC_ROWS)], locp_v[b], sems[b]
            ),
        )

    pending = start_load(0)
    cntvec = jnp.zeros((16,), jnp.int32)
    flushed = jnp.int32(0)

    for s in range(NSUB):
        for cp in pending:
            cp.wait()
        if s + 1 < NSUB:
            pending = start_load(s + 1)
        b = s % 2
        featp = featp_v[b]
        locp = locp_v[b]

        def group(g, cnt):
            foff = 256 * (g // 8) + 16 * (g % 8)
            loff = 512 * (g // 8) + 16 * (g % 8)
            f1 = featp[pl.ds(foff + 128, 16)]
            m = f1 > 0.5
            pos = cnt + plsc.cumsum(m.astype(jnp.int32)) - 1
            rp = pos & (RING - 1)
            f0 = featp[pl.ds(foff, 16)]
            plsc.store_scatter(ringf_v, [rp, zeros_v], f0, mask=m)
            plsc.store_scatter(ringf_v, [rp, ones_v], f1, mask=m)
            for c in range(4):
                cc = jnp.full((16,), c, jnp.int32)
                lv = locp[pl.ds(loff + 128 * c, 16)]
                plsc.store_scatter(ringl_v, [rp, cc], lv, mask=m)
            return cnt + plsc.all_reduce_population_count(m)

        cntvec = lax.fori_loop(0, GPS, group, cntvec, unroll=2)

        cnt_s = jnp.sum(jnp.where(lanes == 0, cntvec, 0))
        nblk = (cnt_s - flushed) // B

        def flush(k, fl):
            roff = fl & (RING - 1)
            pltpu.sync_copy(
                ringf_v.at[pl.ds(roff, B)], outf_hbm.at[pl.ds(w_off + fl, B)]
            )
            pltpu.sync_copy(
                ringl_v.at[pl.ds(roff, B)], outl_hbm.at[pl.ds(w_off + fl, B)]
            )
            return fl + B

        flushed = lax.fori_loop(0, nblk, flush, flushed)

    rem = cnt_w - flushed
    rbase = flushed & (RING - 1)
    dst = w_off + flushed
    off = jnp.int32(0)
    for sz in (256, 128, 64, 32, 16, 8, 4, 2, 1):
        part = rem & sz
        cur = off

        @pl.when(part > 0)
        def _copy(cur=cur, sz=sz):
            pltpu.sync_copy(
                ringf_v.at[pl.ds(rbase + cur, sz)],
                outf_hbm.at[pl.ds(dst + cur, sz)],
            )
            pltpu.sync_copy(
                ringl_v.at[pl.ds(rbase + cur, sz)],
                outl_hbm.at[pl.ds(dst + cur, sz)],
            )

        off = cur + part

    zb = jnp.maximum(0, jnp.minimum(NZB, (nsel - base + ZB - 1) // ZB))

    def zero_body(k, carry):
        row = base + k * ZB
        pltpu.sync_copy(zf_v, outf_hbm.at[pl.ds(row, ZB)])
        pltpu.sync_copy(zi_v, outl_hbm.at[pl.ds(row, ZB)])
        return carry

    lax.fori_loop(zb, NZB, zero_body, 0)

    gap = (ZB - (nsel & (ZB - 1))) & (ZB - 1)
    zoff = jnp.int32(0)
    for zsz in (128, 64, 32, 16, 8, 4, 2, 1):
        zpart = gap & zsz
        zcur = zoff

        @pl.when((zpart > 0) & (nsel >= base) & (nsel < base + C))
        def _zcopy(zcur=zcur, zsz=zsz):
            pltpu.sync_copy(
                zf_v.at[pl.ds(0, zsz)], outf_hbm.at[pl.ds(nsel + zcur, zsz)]
            )
            pltpu.sync_copy(
                zi_v.at[pl.ds(0, zsz)], outl_hbm.at[pl.ds(nsel + zcur, zsz)]
            )

        zoff = zcur + zpart

    @pl.when(w == 0)
    def _nsel():
        nsel_v[...] = jnp.full((16,), nsel, jnp.int32)
        pltpu.sync_copy(nsel_v, nsel_hbm)


TS = 2048
NTS = C // TS
TBL = TS // 128


@functools.partial(
    pl.kernel,
    out_type=(
        jax.ShapeDtypeStruct((2 * N,), jnp.float32),
        jax.ShapeDtypeStruct((4 * N,), jnp.int32),
    ),
    mesh=_mesh,
    compiler_params=_params,
    scratch_types=[
        pltpu.VMEM((TS * 2,), jnp.float32),
        pltpu.VMEM((TS * 2,), jnp.float32),
        pltpu.VMEM((TS * 4,), jnp.int32),
        pltpu.VMEM((TS * 4,), jnp.int32),
    ],
)
def _transpose_kernel(frm_hbm, lrm_hbm, fb_hbm, lb_hbm, fi_v, fo_v, li_v, lo_v):
    w = _wid()
    base = w * C
    lanes = lax.iota(jnp.int32, 16)

    for t in range(NTS):
        row = base + t * TS
        pltpu.sync_copy(frm_hbm.at[pl.ds(2 * row, 2 * TS)], fi_v)
        pltpu.sync_copy(lrm_hbm.at[pl.ds(4 * row, 4 * TS)], li_v)

        def tb(bl, carry):
            fb = 256 * bl
            lb = 512 * bl
            for j in range(8):
                r = 16 * j + lanes
                c0 = plsc.load_gather(fi_v, [fb + r * 2])
                c1 = plsc.load_gather(fi_v, [fb + r * 2 + 1])
                fo_v[pl.ds(fb + 16 * j, 16)] = c0
                fo_v[pl.ds(fb + 128 + 16 * j, 16)] = c1
                for c in range(4):
                    lv = plsc.load_gather(li_v, [lb + r * 4 + c])
                    lo_v[pl.ds(lb + 128 * c + 16 * j, 16)] = lv
            return carry

        lax.fori_loop(0, TBL, tb, 0)
        pltpu.sync_copy(fo_v, fb_hbm.at[pl.ds(2 * row, 2 * TS)])
        pltpu.sync_copy(lo_v, lb_hbm.at[pl.ds(4 * row, 4 * TS)])


@functools.lru_cache(maxsize=1)
def _jitted_kernel():
    dev = jax.devices()[0]
    sharding = jax.sharding.SingleDeviceSharding(dev)
    rm2 = jex_layout.Format(
        jex_layout.Layout(major_to_minor=(0, 1)), sharding
    )
    sc = jex_layout.Format(jex_layout.Layout(major_to_minor=()), sharding)
    return jax.jit(
        _kernel_impl,
        in_shardings=(rm2, rm2),
        out_shardings=(rm2, rm2, sc),
    )


def kernel(features, locations):
    return _jitted_kernel()(features, locations)


def _kernel_impl(features, locations):
    locations = locations.astype(jnp.int32)
    feat_b = features.reshape(N // 128, 128, 2).transpose(0, 2, 1).reshape(-1)
    loc_b = locations.reshape(N // 128, 128, 4).transpose(0, 2, 1).reshape(-1)
    counts = _count_kernel(feat_b)
    zerof = jnp.zeros((ZB, 2), jnp.float32)
    zeroi = jnp.zeros((ZB, 4), jnp.int32)
    outf, outl, nsel = _compact_kernel(
        feat_b, loc_b, counts, zerof, zeroi
    )
    fb, lb = _transpose_kernel(outf.reshape(-1), outl.reshape(-1))
    sel_f = fb.reshape(N // 128, 2, 128).transpose(0, 2, 1).reshape(N, 2)
    sel_l = lb.reshape(N // 128, 4, 128).transpose(0, 2, 1).reshape(N, 4)
    return sel_f, sel_l, nsel[0]
